# baseline (speedup 1.0000x reference)
import functools

import jax
import jax.numpy as jnp
from jax import lax
from jax.experimental import pallas as pl
from jax.experimental.pallas import tpu as pltpu
from jax.experimental.pallas import tpu_sc as plsc

LANES = 16
BLK = 128
CB = 20


@functools.cache
def _build_call(n_nodes: int, n_edges: int):
    num_cores, num_subcores = 2, 16
    nw = num_cores * num_subcores
    n_nodes_pad = (n_nodes + 127) // 128 * 128
    chunk = CB * BLK
    rows_per_chunk = 2 * CB
    assert n_edges % chunk == 0
    n_global = n_edges // chunk
    n_full = n_global // nw
    n_extra = n_global % nw
    assert n_full >= 2
    n_inner = chunk // LANES

    mesh = plsc.VectorSubcoreMesh(
        core_axis_name="c", subcore_axis_name="s",
        num_cores=num_cores, num_subcores=num_subcores,
    )

    @functools.partial(
        pl.kernel,
        out_type=jax.ShapeDtypeStruct((n_edges,), jnp.float32),
        mesh=mesh,
        compiler_params=pltpu.CompilerParams(needs_layout_passes=False),
        scratch_types=[
            pltpu.VMEM((n_nodes_pad,), jnp.int32),
            pltpu.VMEM((128,), jnp.float32),
            pltpu.VMEM((128,), jnp.float32),
            pltpu.VMEM((128,), jnp.float32),
            pltpu.VMEM((rows_per_chunk, BLK), jnp.int32),
            pltpu.VMEM((rows_per_chunk, BLK), jnp.int32),
            pltpu.VMEM((chunk,), jnp.float32),
            pltpu.VMEM((chunk,), jnp.float32),
            pltpu.SemaphoreType.DMA((2,)),
            pltpu.SemaphoreType.DMA((2,)),
        ],
    )
    def _sc_bias(types_hbm, edge_hbm, imp_hbm, cross_hbm, out_hbm,
                 types_v, imp_v, cross_v, table_v, ebuf0, ebuf1,
                 out_v0, out_v1, isem, osem):
        ebufs = (ebuf0, ebuf1)
        outs = (out_v0, out_v1)
        wid = lax.axis_index("s") * num_cores + lax.axis_index("c")
        n_my = n_full + jnp.where(wid < n_extra, 1, 0)

        pltpu.sync_copy(types_hbm, types_v)
        pltpu.sync_copy(imp_hbm, imp_v)
        pltpu.sync_copy(cross_hbm, cross_v)

        iota = lax.iota(jnp.int32, LANES)
        for k in range(2):
            idx = iota + LANES * k
            a = lax.shift_right_logical(idx * 13, 6)
            b = idx - a * 5
            va = plsc.load_gather(imp_v, [a])
            vb = plsc.load_gather(imp_v, [b])
            vc = plsc.load_gather(cross_v, [idx])
            table_v[pl.ds(LANES * k, LANES)] = va + vb + vc

        def in_copy(i, b):
            gidx = i * nw + wid
            return pltpu.make_async_copy(
                edge_hbm.at[pl.ds(gidx * rows_per_chunk, rows_per_chunk), :],
                ebufs[b], isem.at[b])

        def out_copy(i, b):
            gidx = i * nw + wid
            return pltpu.make_async_copy(
                outs[b], out_hbm.at[pl.ds(gidx * chunk, chunk)], osem.at[b])

        def compute(b):
            ebuf, out_v = ebufs[b], outs[b]

            @plsc.parallel_loop(0, CB, step=1, unroll=2)
            def _(j):
                j2 = j * 2
                obase = j * BLK
                for c in range(BLK // LANES):
                    off = c * LANES
                    r16 = ebuf[j2, pl.ds(off, LANES)]
                    c16 = ebuf[j2 + 1, pl.ds(off, LANES)]
                    ti = plsc.load_gather(types_v, [r16])
                    tj = plsc.load_gather(types_v, [c16])
                    comb = ti * 5 + tj
                    out_v[pl.ds(obase + off, LANES)] = plsc.load_gather(
                        table_v, [comb])

        in_copy(0, 0).start()
        in_copy(1, 1).start()

        def body(i, carry):
            for b in range(2):
                @pl.when((i & 1) == b)
                def _():
                    in_copy(i, b).wait()

                    @pl.when(i >= 2)
                    def _():
                        out_copy(i - 2, b).wait()

                    compute(b)
                    out_copy(i, b).start()

                    @pl.when(i + 2 < n_my)
                    def _():
                        in_copy(i + 2, b).start()

            return carry

        lax.fori_loop(0, n_my, body, 0)
        out_copy(0, 0).wait()
        out_copy(1, 1).wait()

    return _sc_bias


def kernel(motif_types, edge_index, motif_importance, cross_motif_bias):
    n_nodes = motif_types.shape[0]
    n_edges = edge_index.shape[1]
    num_types = motif_importance.shape[0]
    n_nodes_pad = (n_nodes + 127) // 128 * 128
    nblk = n_edges // BLK
    edges = (
        edge_index.astype(jnp.int32)
        .reshape(2, nblk, BLK)
        .transpose(1, 0, 2)
        .reshape(2 * nblk, BLK)
    )
    types = jnp.pad(motif_types.astype(jnp.int32), (0, n_nodes_pad - n_nodes))
    imp = jnp.pad(motif_importance.astype(jnp.float32), (0, 128 - num_types))
    cross = jnp.pad(
        cross_motif_bias.astype(jnp.float32).reshape(-1),
        (0, 128 - num_types * num_types),
    )
    return _build_call(n_nodes, n_edges)(types, edges, imp, cross)


# speedup vs baseline: 1.2488x; 1.2488x over previous
import functools

import jax
import jax.numpy as jnp
from jax import lax
from jax.experimental import pallas as pl
from jax.experimental.pallas import tpu as pltpu
from jax.experimental.pallas import tpu_sc as plsc

LANES = 16
BLK = 128
CB = 40


@functools.cache
def _build_call(n_nodes: int, n_edges: int):
    num_cores, num_subcores = 2, 16
    nw = num_cores * num_subcores
    n_nodes_pad = (n_nodes + 7) // 8 * 8
    chunk = CB * BLK
    rows_per_chunk = 2 * CB
    assert n_edges % chunk == 0
    n_global = n_edges // chunk
    n_full = n_global // nw
    n_extra = n_global % nw
    assert n_full >= 2
    n_inner = chunk // LANES

    mesh = plsc.VectorSubcoreMesh(
        core_axis_name="c", subcore_axis_name="s",
        num_cores=num_cores, num_subcores=num_subcores,
    )

    @functools.partial(
        pl.kernel,
        out_type=jax.ShapeDtypeStruct((n_edges,), jnp.float32),
        mesh=mesh,
        compiler_params=pltpu.CompilerParams(needs_layout_passes=False),
        scratch_types=[
            pltpu.VMEM((n_nodes_pad,), jnp.int32),
            pltpu.VMEM((96,), jnp.float32),
            pltpu.VMEM((rows_per_chunk, BLK), jnp.int32),
            pltpu.VMEM((rows_per_chunk, BLK), jnp.int32),
            pltpu.VMEM((chunk,), jnp.float32),
            pltpu.VMEM((chunk,), jnp.float32),
            pltpu.SemaphoreType.DMA((2,)),
            pltpu.SemaphoreType.DMA((2,)),
        ],
    )
    def _sc_bias(types_hbm, edge_hbm, imp_hbm, cross_hbm, out_hbm,
                 types_v, param_v, ebuf0, ebuf1,
                 out_v0, out_v1, isem, osem):
        ebufs = (ebuf0, ebuf1)
        outs = (out_v0, out_v1)
        wid = lax.axis_index("s") * num_cores + lax.axis_index("c")
        n_my = n_full + jnp.where(wid < n_extra, 1, 0)

        pltpu.sync_copy(types_hbm, types_v)
        pltpu.sync_copy(imp_hbm, param_v.at[pl.ds(0, 16)])
        pltpu.sync_copy(cross_hbm, param_v.at[pl.ds(16, 32)])

        iota = lax.iota(jnp.int32, LANES)
        for k in range(2):
            idx = iota + LANES * k
            a = lax.shift_right_logical(idx * 13, 6)
            b = idx - a * 5
            va = plsc.load_gather(param_v, [a])
            vb = plsc.load_gather(param_v, [b])
            vc = plsc.load_gather(param_v, [idx + 16])
            param_v[pl.ds(48 + LANES * k, LANES)] = va + vb + vc

        def in_copy(i, b):
            gidx = i * nw + wid
            return pltpu.make_async_copy(
                edge_hbm.at[pl.ds(gidx * rows_per_chunk, rows_per_chunk), :],
                ebufs[b], isem.at[b])

        def out_copy(i, b):
            gidx = i * nw + wid
            return pltpu.make_async_copy(
                outs[b], out_hbm.at[pl.ds(gidx * chunk, chunk)], osem.at[b])

        def compute(b):
            ebuf, out_v = ebufs[b], outs[b]

            @plsc.parallel_loop(0, n_inner, step=1, unroll=8)
            def _(i):
                j2 = (i >> 3) * 2
                c = (i & 7) * LANES
                r16 = ebuf[j2, pl.ds(c, LANES)]
                c16 = ebuf[j2 + 1, pl.ds(c, LANES)]
                ti = plsc.load_gather(types_v, [r16])
                tj = plsc.load_gather(types_v, [c16])
                comb = ti * 5 + (tj + 48)
                out_v[pl.ds(i * LANES, LANES)] = plsc.load_gather(
                    param_v, [comb])

        in_copy(0, 0).start()
        in_copy(1, 1).start()

        def body(i, carry):
            for b in range(2):
                @pl.when((i & 1) == b)
                def _():
                    in_copy(i, b).wait()

                    @pl.when(i >= 2)
                    def _():
                        out_copy(i - 2, b).wait()

                    compute(b)
                    out_copy(i, b).start()

                    @pl.when(i + 2 < n_my)
                    def _():
                        in_copy(i + 2, b).start()

            return carry

        lax.fori_loop(0, n_my, body, 0)
        out_copy(0, 0).wait()
        out_copy(1, 1).wait()

    return _sc_bias


def kernel(motif_types, edge_index, motif_importance, cross_motif_bias):
    n_nodes = motif_types.shape[0]
    n_edges = edge_index.shape[1]
    num_types = motif_importance.shape[0]
    n_nodes_pad = (n_nodes + 7) // 8 * 8
    nblk = n_edges // BLK
    edges = (
        edge_index.astype(jnp.int32)
        .reshape(2, nblk, BLK)
        .transpose(1, 0, 2)
        .reshape(2 * nblk, BLK)
    )
    types = jnp.pad(motif_types.astype(jnp.int32), (0, n_nodes_pad - n_nodes))
    imp = jnp.pad(motif_importance.astype(jnp.float32), (0, 16 - num_types))
    cross = jnp.pad(
        cross_motif_bias.astype(jnp.float32).reshape(-1),
        (0, 32 - num_types * num_types),
    )
    return _build_call(n_nodes, n_edges)(types, edges, imp, cross)


# speedup vs baseline: 1.2552x; 1.0051x over previous
import functools

import jax
import jax.numpy as jnp
from jax import lax
from jax.experimental import pallas as pl
from jax.experimental.pallas import tpu as pltpu
from jax.experimental.pallas import tpu_sc as plsc

LANES = 16
BLK = 128
CB = 40


@functools.cache
def _build_call(n_nodes: int, n_edges: int):
    num_cores, num_subcores = 2, 16
    nw = num_cores * num_subcores
    n_nodes_pad = (n_nodes + 7) // 8 * 8
    chunk = CB * BLK
    rows_per_chunk = 2 * CB
    assert n_edges % chunk == 0
    n_global = n_edges // chunk
    n_full = n_global // nw
    n_extra = n_global % nw
    assert n_full >= 2
    n_inner = chunk // LANES

    mesh = plsc.VectorSubcoreMesh(
        core_axis_name="c", subcore_axis_name="s",
        num_cores=num_cores, num_subcores=num_subcores,
    )

    @functools.partial(
        pl.kernel,
        out_type=jax.ShapeDtypeStruct((n_edges,), jnp.float32),
        mesh=mesh,
        compiler_params=pltpu.CompilerParams(needs_layout_passes=False),
        scratch_types=[
            pltpu.VMEM((n_nodes_pad,), jnp.int32),
            pltpu.VMEM((96,), jnp.float32),
            pltpu.VMEM((rows_per_chunk, BLK), jnp.int32),
            pltpu.VMEM((rows_per_chunk, BLK), jnp.int32),
            pltpu.VMEM((chunk,), jnp.float32),
            pltpu.VMEM((chunk,), jnp.float32),
            pltpu.SemaphoreType.DMA((2,)),
            pltpu.SemaphoreType.DMA((2,)),
        ],
    )
    def _sc_bias(types_hbm, edge_hbm, aux_hbm, out_hbm,
                 types_v, param_v, ebuf0, ebuf1,
                 out_v0, out_v1, isem, osem):
        ebufs = (ebuf0, ebuf1)
        outs = (out_v0, out_v1)
        wid = lax.axis_index("s") * num_cores + lax.axis_index("c")
        n_my = n_full + jnp.where(wid < n_extra, 1, 0)

        pltpu.sync_copy(types_hbm, types_v)
        pltpu.sync_copy(aux_hbm, param_v.at[pl.ds(0, 48)])

        iota = lax.iota(jnp.int32, LANES)
        for k in range(2):
            idx = iota + LANES * k
            a = lax.shift_right_logical(idx * 13, 6)
            b = idx - a * 5
            va = plsc.load_gather(param_v, [a])
            vb = plsc.load_gather(param_v, [b])
            vc = plsc.load_gather(param_v, [idx + 5])
            param_v[pl.ds(48 + LANES * k, LANES)] = va + vb + vc

        def in_copy(i, b):
            gidx = i * nw + wid
            return pltpu.make_async_copy(
                edge_hbm.at[pl.ds(gidx * rows_per_chunk, rows_per_chunk), :],
                ebufs[b], isem.at[b])

        def out_copy(i, b):
            gidx = i * nw + wid
            return pltpu.make_async_copy(
                outs[b], out_hbm.at[pl.ds(gidx * chunk, chunk)], osem.at[b])

        def compute(b):
            ebuf, out_v = ebufs[b], outs[b]

            @plsc.parallel_loop(0, n_inner, step=1, unroll=8)
            def _(i):
                j2 = (i >> 3) * 2
                c = (i & 7) * LANES
                r16 = ebuf[j2, pl.ds(c, LANES)]
                c16 = ebuf[j2 + 1, pl.ds(c, LANES)]
                ti = plsc.load_gather(types_v, [r16])
                tj = plsc.load_gather(types_v, [c16])
                comb = ti * 5 + (tj + 48)
                out_v[pl.ds(i * LANES, LANES)] = plsc.load_gather(
                    param_v, [comb])

        in_copy(0, 0).start()
        in_copy(1, 1).start()

        def body(i, carry):
            for b in range(2):
                @pl.when((i & 1) == b)
                def _():
                    in_copy(i, b).wait()

                    @pl.when(i >= 2)
                    def _():
                        out_copy(i - 2, b).wait()

                    compute(b)
                    out_copy(i, b).start()

                    @pl.when(i + 2 < n_my)
                    def _():
                        in_copy(i + 2, b).start()

            return carry

        lax.fori_loop(0, n_my, body, 0)
        out_copy(0, 0).wait()
        out_copy(1, 1).wait()

    return _sc_bias


def kernel(motif_types, edge_index, motif_importance, cross_motif_bias):
    n_nodes = motif_types.shape[0]
    n_edges = edge_index.shape[1]
    num_types = motif_importance.shape[0]
    n_nodes_pad = (n_nodes + 7) // 8 * 8
    nblk = n_edges // BLK
    edges = (
        edge_index.astype(jnp.int32)
        .reshape(2, nblk, BLK)
        .transpose(1, 0, 2)
        .reshape(2 * nblk, BLK)
    )
    types = jnp.pad(motif_types.astype(jnp.int32), (0, n_nodes_pad - n_nodes))
    aux = jnp.pad(
        jnp.concatenate([
            motif_importance.astype(jnp.float32),
            cross_motif_bias.astype(jnp.float32).reshape(-1),
        ]),
        (0, 48 - num_types - num_types * num_types),
    )
    return _build_call(n_nodes, n_edges)(types, edges, aux)


# speedup vs baseline: 1.2701x; 1.0119x over previous
import functools

import jax
import jax.numpy as jnp
from jax import lax
from jax.experimental import pallas as pl
from jax.experimental.pallas import tpu as pltpu
from jax.experimental.pallas import tpu_sc as plsc

LANES = 16
BLK = 128
CB = 40


@functools.cache
def _build_call(n_nodes: int, n_edges: int):
    num_cores, num_subcores = 2, 16
    nw = num_cores * num_subcores
    n_nodes_pad = (n_nodes + 7) // 8 * 8
    chunk = CB * BLK
    rows_per_chunk = 2 * CB
    assert n_edges % chunk == 0
    n_global = n_edges // chunk
    n_full = n_global // nw
    n_extra = n_global % nw
    assert n_full >= 2
    n_inner = chunk // LANES

    mesh = plsc.VectorSubcoreMesh(
        core_axis_name="c", subcore_axis_name="s",
        num_cores=num_cores, num_subcores=num_subcores,
    )

    @functools.partial(
        pl.kernel,
        out_type=jax.ShapeDtypeStruct((n_edges,), jnp.float32),
        mesh=mesh,
        compiler_params=pltpu.CompilerParams(needs_layout_passes=False),
        scratch_types=[
            pltpu.VMEM((n_nodes_pad,), jnp.int32),
            pltpu.VMEM((96,), jnp.float32),
            pltpu.VMEM((rows_per_chunk, BLK), jnp.int32),
            pltpu.VMEM((rows_per_chunk, BLK), jnp.int32),
            pltpu.VMEM((chunk,), jnp.float32),
            pltpu.VMEM((chunk,), jnp.float32),
            pltpu.SemaphoreType.DMA((2,)),
            pltpu.SemaphoreType.DMA((2,)),
        ],
    )
    def _sc_bias(types_hbm, edge_hbm, aux_hbm, out_hbm,
                 types_v, param_v, ebuf0, ebuf1,
                 out_v0, out_v1, isem, osem):
        ebufs = (ebuf0, ebuf1)
        outs = (out_v0, out_v1)
        wid = lax.axis_index("s") * num_cores + lax.axis_index("c")
        n_my = n_full + jnp.where(wid < n_extra, 1, 0)

        def in_copy(i, b):
            gidx = i * nw + wid
            return pltpu.make_async_copy(
                edge_hbm.at[pl.ds(gidx * rows_per_chunk, rows_per_chunk), :],
                ebufs[b], isem.at[b])

        def out_copy(i, b):
            gidx = i * nw + wid
            return pltpu.make_async_copy(
                outs[b], out_hbm.at[pl.ds(gidx * chunk, chunk)], osem.at[b])

        in_copy(0, 0).start()
        in_copy(1, 1).start()

        pltpu.sync_copy(types_hbm, types_v)
        pltpu.sync_copy(aux_hbm, param_v.at[pl.ds(0, 48)])

        iota = lax.iota(jnp.int32, LANES)
        for k in range(2):
            idx = iota + LANES * k
            a = lax.shift_right_logical(idx * 13, 6)
            b = idx - a * 5
            va = plsc.load_gather(param_v, [a])
            vb = plsc.load_gather(param_v, [b])
            vc = plsc.load_gather(param_v, [idx + 5])
            param_v[pl.ds(48 + LANES * k, LANES)] = va + vb + vc

        def compute(b):
            ebuf, out_v = ebufs[b], outs[b]

            @plsc.parallel_loop(0, n_inner, step=1, unroll=8)
            def _(i):
                j2 = (i >> 3) * 2
                c = (i & 7) * LANES
                r16 = ebuf[j2, pl.ds(c, LANES)]
                c16 = ebuf[j2 + 1, pl.ds(c, LANES)]
                ti = plsc.load_gather(types_v, [r16])
                tj = plsc.load_gather(types_v, [c16])
                comb = ti * 5 + (tj + 48)
                out_v[pl.ds(i * LANES, LANES)] = plsc.load_gather(
                    param_v, [comb])

        def body(i, carry):
            for b in range(2):
                @pl.when((i & 1) == b)
                def _():
                    in_copy(i, b).wait()

                    @pl.when(i >= 2)
                    def _():
                        out_copy(i - 2, b).wait()

                    compute(b)
                    out_copy(i, b).start()

                    @pl.when(i + 2 < n_my)
                    def _():
                        in_copy(i + 2, b).start()

            return carry

        lax.fori_loop(0, n_my, body, 0)
        out_copy(0, 0).wait()
        out_copy(1, 1).wait()

    return _sc_bias


def kernel(motif_types, edge_index, motif_importance, cross_motif_bias):
    n_nodes = motif_types.shape[0]
    n_edges = edge_index.shape[1]
    num_types = motif_importance.shape[0]
    n_nodes_pad = (n_nodes + 7) // 8 * 8
    nblk = n_edges // BLK
    edges = (
        edge_index.astype(jnp.int32)
        .reshape(2, nblk, BLK)
        .transpose(1, 0, 2)
        .reshape(2 * nblk, BLK)
    )
    types = jnp.pad(motif_types.astype(jnp.int32), (0, n_nodes_pad - n_nodes))
    aux = jnp.pad(
        jnp.concatenate([
            motif_importance.astype(jnp.float32),
            cross_motif_bias.astype(jnp.float32).reshape(-1),
        ]),
        (0, 48 - num_types - num_types * num_types),
    )
    return _build_call(n_nodes, n_edges)(types, edges, aux)
